# baseline (speedup 1.0000x reference)
import functools

import jax
import jax.numpy as jnp
from jax import lax
from jax.experimental import pallas as pl
from jax.experimental.pallas import tpu as pltpu
from jax.experimental.pallas import tpu_sc as plsc

N = 10000
E = 320000
D = 128

NC = 2
NS = 16
NW = NC * NS
EPW = E // NW

EPWP = 10240
PB = 128
NPB = EPWP // PB

NHALF = 5000
NPH = 5120
PRC = 4
PRL = 80


def _mesh():
    return plsc.VectorSubcoreMesh(core_axis_name="c", subcore_axis_name="s")


def _identity_rows(idref, base, rc, rcl):
    i16 = lax.iota(jnp.int32, 16)

    def ib(t, carry):
        c = t // (rcl // 16)
        k = t % (rcl // 16)
        idref[c, pl.ds(k * 16, 16)] = base + c * rcl + k * 16 + i16
        return carry

    lax.fori_loop(0, rc * (rcl // 16), ib, 0)


def _make_hist(half):
    lo = half * NHALF

    @functools.partial(
        pl.kernel,
        out_type=jax.ShapeDtypeStruct((NW * PRC, PRL, D), jnp.float32),
        scratch_types=[
            pltpu.VMEM((NPB, PB), jnp.int32),
            pltpu.VMEM((PB, D), jnp.float32),
            pltpu.VMEM((PRL, D), jnp.float32),
            pltpu.VMEM((PRC, PRL), jnp.int32),
            pltpu.VMEM_SHARED((NPH, D), jnp.float32),
            pltpu.SemaphoreType.DMA,
        ],
        mesh=_mesh(),
    )
    def hist(dst_ref, hp_ref, dstv, onesv, stg, idr, acc, ssem):
        cid = lax.axis_index("c")
        sid = lax.axis_index("s")
        w = cid * NS + sid
        pltpu.sync_copy(dst_ref.at[w], dstv)
        z16 = jnp.zeros((16,), jnp.float32)
        ones16 = jnp.ones((16,), jnp.float32)

        def tb(t, carry):
            j = t // (PB // 16)
            k = t % (PB // 16)
            sl = pl.ds(k * 16, 16)
            d = dstv[j, sl]
            dl = d - lo
            inh = (dl >= 0) & (dl < NHALF)
            dstv[j, sl] = jnp.where(inh, dl, NHALF + (d & 63))
            return carry

        lax.fori_loop(0, NPB * (PB // 16), tb, 0)

        def ob(i, carry):
            def oc(k, carry2):
                onesv[i, pl.ds(k * 16, 16)] = ones16
                return carry2

            lax.fori_loop(0, D // 16, oc, 0)
            return carry

        lax.fori_loop(0, PB, ob, 0)

        def zb(i, carry):
            def zc(k, carry2):
                stg[i, pl.ds(k * 16, 16)] = z16
                return carry2

            lax.fori_loop(0, D // 16, zc, 0)
            return carry

        lax.fori_loop(0, PRL, zb, 0)
        _identity_rows(idr, sid * (PRC * PRL), PRC, PRL)

        def zs(c, carry):
            pltpu.sync_copy(stg, acc.at[idr.at[c]])
            return carry

        lax.fori_loop(0, PRC, zs, 0)
        plsc.subcore_barrier()

        def hb(j, carry):
            pltpu.async_copy(onesv, acc.at[dstv.at[j]], ssem, add=True)
            return carry

        lax.fori_loop(0, NPB, hb, 0)

        def dr(j, carry):
            pltpu.make_async_copy(onesv, acc.at[dstv.at[j]], ssem).wait()
            return carry

        lax.fori_loop(0, NPB, dr, 0)
        plsc.subcore_barrier()

        def rb(c, carry):
            pltpu.sync_copy(acc.at[idr.at[c]], stg)
            pltpu.sync_copy(stg, hp_ref.at[w * PRC + c])
            return carry

        lax.fori_loop(0, PRC, rb, 0)

    return hist


_hist0 = _make_hist(0)
_hist1 = _make_hist(1)


def _make_pool(half):
    lo = half * NHALF

    @functools.partial(
        pl.kernel,
        out_type=jax.ShapeDtypeStruct((NW * PRC, PRL, D), jnp.float32),
        scratch_types=[
            pltpu.VMEM((NPB, PB), jnp.int32),
            pltpu.VMEM((NPB, PB), jnp.int32),
            pltpu.VMEM((PB, D), jnp.float32),
            pltpu.VMEM((PB, D), jnp.float32),
            pltpu.VMEM((PRL, D), jnp.float32),
            pltpu.VMEM((PRC, PRL), jnp.int32),
            pltpu.VMEM_SHARED((NPH, D), jnp.float32),
            pltpu.SemaphoreType.DMA,
            pltpu.SemaphoreType.DMA,
            pltpu.SemaphoreType.DMA,
            pltpu.SemaphoreType.DMA,
        ],
        mesh=_mesh(),
    )
    def pool(xn_ref, src_ref, dst_ref, p_ref, srcv, dstv, rbuf0, rbuf1, stg,
             idr, acc, gsem, gsem1, ssem0, ssem1):
        cid = lax.axis_index("c")
        sid = lax.axis_index("s")
        w = cid * NS + sid
        pltpu.sync_copy(src_ref.at[w], srcv)
        pltpu.sync_copy(dst_ref.at[w], dstv)
        z16 = jnp.zeros((16,), jnp.float32)

        def tb(t, carry):
            j = t // (PB // 16)
            k = t % (PB // 16)
            sl = pl.ds(k * 16, 16)
            d = dstv[j, sl]
            dl = d - lo
            inh = (dl >= 0) & (dl < NHALF)
            dstv[j, sl] = jnp.where(inh, dl, NHALF + (d & 63))
            return carry

        lax.fori_loop(0, NPB * (PB // 16), tb, 0)

        def zb(i, carry):
            def zc(k, carry2):
                stg[i, pl.ds(k * 16, 16)] = z16
                return carry2

            lax.fori_loop(0, D // 16, zc, 0)
            return carry

        lax.fori_loop(0, PRL, zb, 0)
        _identity_rows(idr, sid * (PRC * PRL), PRC, PRL)

        def zs(c, carry):
            pltpu.sync_copy(stg, acc.at[idr.at[c]])
            return carry

        lax.fori_loop(0, PRC, zs, 0)
        plsc.subcore_barrier()

        def step(j, rb_t, gsem_t, ssem_t, rb_o, gsem_o, ssem_o):
            pltpu.make_async_copy(xn_ref.at[srcv.at[j]], rb_t, gsem_t).wait()
            pltpu.async_copy(rb_t, acc.at[dstv.at[j]], ssem_t, add=True)

            @pl.when(j >= 1)
            def _():
                pltpu.make_async_copy(rb_o, acc.at[idr.at[0]], ssem_o).wait()

            @pl.when(j + 1 < NPB)
            def _():
                pltpu.async_copy(xn_ref.at[srcv.at[j + 1]], rb_o, gsem_o)

        pltpu.async_copy(xn_ref.at[srcv.at[0]], rbuf0, gsem)

        def body(j, carry):
            @pl.when((j & 1) == 0)
            def _():
                step(j, rbuf0, gsem, ssem0, rbuf1, gsem1, ssem1)

            @pl.when((j & 1) == 1)
            def _():
                step(j, rbuf1, gsem1, ssem1, rbuf0, gsem, ssem0)

            return carry

        lax.fori_loop(0, NPB, body, 0)
        pltpu.make_async_copy(rbuf1, acc.at[idr.at[0]], ssem1).wait()
        plsc.subcore_barrier()

        def rb(c, carry):
            pltpu.sync_copy(acc.at[idr.at[c]], stg)
            pltpu.sync_copy(stg, p_ref.at[w * PRC + c])
            return carry

        lax.fori_loop(0, PRC, rb, 0)

    return pool


_pool0 = _make_pool(0)
_pool1 = _make_pool(1)


def _invd_body(h00_ref, h01_ref, h10_ref, h11_ref, o_ref):
    d0 = h00_ref[...][:NHALF, :1] + h01_ref[...][:NHALF, :1]
    d1 = h10_ref[...][:NHALF, :1] + h11_ref[...][:NHALF, :1]
    o_ref[...] = lax.rsqrt(jnp.concatenate([d0, d1], axis=0))


def _invd_call(h00, h01, h10, h11):
    return pl.pallas_call(
        _invd_body,
        out_shape=jax.ShapeDtypeStruct((N, 1), jnp.float32),
    )(h00, h01, h10, h11)


def _scale_body(x_ref, iv_ref, o_ref):
    o_ref[...] = x_ref[...] * iv_ref[...]


def _out_body(p0_ref, p1_ref, iv_ref, w_ref, b_ref, o_ref):
    pooled = (p0_ref[...] + p1_ref[...]) * iv_ref[...]
    acc = jnp.dot(pooled, w_ref[...], preferred_element_type=jnp.float32)
    o_ref[...] = jnp.maximum(acc + b_ref[...], 0.0)


_RB = 2000


def _scale_call(x, iv):
    return pl.pallas_call(
        _scale_body,
        grid=(N // _RB,),
        in_specs=[
            pl.BlockSpec((_RB, D), lambda i: (i, 0)),
            pl.BlockSpec((_RB, 1), lambda i: (i, 0)),
        ],
        out_specs=pl.BlockSpec((_RB, D), lambda i: (i, 0)),
        out_shape=jax.ShapeDtypeStruct((N, D), jnp.float32),
    )(x, iv)


def _out_call(p0, p1, iv, W, b2):
    return pl.pallas_call(
        _out_body,
        grid=(N // _RB,),
        in_specs=[
            pl.BlockSpec((_RB, D), lambda i: (i, 0)),
            pl.BlockSpec((_RB, D), lambda i: (i, 0)),
            pl.BlockSpec((_RB, 1), lambda i: (i, 0)),
            pl.BlockSpec((D, D), lambda i: (0, 0)),
            pl.BlockSpec((1, D), lambda i: (0, 0)),
        ],
        out_specs=pl.BlockSpec((_RB, D), lambda i: (i, 0)),
        out_shape=jax.ShapeDtypeStruct((N, D), jnp.float32),
    )(p0, p1, iv, W, b2)


def kernel(x, edge_index, W, b):
    src = edge_index[0].astype(jnp.int32)
    dst = edge_index[1].astype(jnp.int32)
    pad = EPWP - EPW
    src_p = jnp.pad(src.reshape(NW, EPW), ((0, 0), (0, pad))
                    ).reshape(NW, NPB, PB)
    dst_p = jnp.pad(dst.reshape(NW, EPW), ((0, 0), (0, pad)),
                    constant_values=-1).reshape(NW, NPB, PB)

    hh0 = _hist0(dst_p).reshape(NC, NPH, D)
    hh1 = _hist1(dst_p).reshape(NC, NPH, D)
    iv = _invd_call(hh0[0], hh0[1], hh1[0], hh1[1])

    xn = _scale_call(x, iv)

    ph0 = _pool0(xn, src_p, dst_p).reshape(NC, NPH, D)
    ph1 = _pool1(xn, src_p, dst_p).reshape(NC, NPH, D)
    p0 = jnp.concatenate([ph0[0, :NHALF], ph1[0, :NHALF]], axis=0)
    p1 = jnp.concatenate([ph0[1, :NHALF], ph1[1, :NHALF]], axis=0)
    out = _out_call(p0, p1, iv, W, b.reshape(1, D))
    return out


# speedup vs baseline: 2.0144x; 2.0144x over previous
import functools

import jax
import jax.numpy as jnp
from jax import lax
from jax.experimental import pallas as pl
from jax.experimental.pallas import tpu as pltpu
from jax.experimental.pallas import tpu_sc as plsc

N = 10000
E = 320000
D = 128

NC = 2
NS = 16
NW = NC * NS
EPW = E // NW

PB = 80
NPB = EPW // PB

NHALF = 5000
NPH = 5120
PRC = 4
PRL = 80


def _mesh():
    return plsc.VectorSubcoreMesh(core_axis_name="c", subcore_axis_name="s")


def _identity_rows(idref, base, rc, rcl):
    i16 = lax.iota(jnp.int32, 16)

    def ib(t, carry):
        c = t // (rcl // 16)
        k = t % (rcl // 16)
        idref[c, pl.ds(k * 16, 16)] = base + c * rcl + k * 16 + i16
        return carry

    lax.fori_loop(0, rc * (rcl // 16), ib, 0)


def _make_hist(half):
    lo = half * NHALF

    @functools.partial(
        pl.kernel,
        out_type=jax.ShapeDtypeStruct((NW * PRC, PRL, D), jnp.float32),
        scratch_types=[
            pltpu.VMEM((NPB, PB), jnp.int32),
            pltpu.VMEM((PB, D), jnp.float32),
            pltpu.VMEM((PRL, D), jnp.float32),
            pltpu.VMEM((PRC, PRL), jnp.int32),
            pltpu.VMEM_SHARED((NPH, D), jnp.float32),
            pltpu.SemaphoreType.DMA,
        ],
        mesh=_mesh(),
    )
    def hist(dst_ref, hp_ref, dstv, onesv, stg, idr, acc, ssem):
        cid = lax.axis_index("c")
        sid = lax.axis_index("s")
        w = cid * NS + sid
        pltpu.sync_copy(dst_ref.at[w], dstv)
        z16 = jnp.zeros((16,), jnp.float32)
        ones16 = jnp.ones((16,), jnp.float32)

        def tb(t, carry):
            j = t // (PB // 16)
            k = t % (PB // 16)
            sl = pl.ds(k * 16, 16)
            d = dstv[j, sl]
            dl = d - lo
            inh = (dl >= 0) & (dl < NHALF)
            dstv[j, sl] = jnp.where(inh, dl, NHALF + (d & 63))
            return carry

        lax.fori_loop(0, NPB * (PB // 16), tb, 0)

        def ob(i, carry):
            def oc(k, carry2):
                onesv[i, pl.ds(k * 16, 16)] = ones16
                return carry2

            lax.fori_loop(0, D // 16, oc, 0)
            return carry

        lax.fori_loop(0, PB, ob, 0)

        def zb(i, carry):
            def zc(k, carry2):
                stg[i, pl.ds(k * 16, 16)] = z16
                return carry2

            lax.fori_loop(0, D // 16, zc, 0)
            return carry

        lax.fori_loop(0, PRL, zb, 0)
        _identity_rows(idr, sid * (PRC * PRL), PRC, PRL)

        def zs(c, carry):
            pltpu.sync_copy(stg, acc.at[idr.at[c]])
            return carry

        lax.fori_loop(0, PRC, zs, 0)
        plsc.subcore_barrier()

        def hb(j, carry):
            pltpu.async_copy(onesv, acc.at[dstv.at[j]], ssem, add=True)
            return carry

        lax.fori_loop(0, NPB, hb, 0)

        def dr(j, carry):
            pltpu.make_async_copy(onesv, acc.at[dstv.at[j]], ssem).wait()
            return carry

        lax.fori_loop(0, NPB, dr, 0)
        plsc.subcore_barrier()

        def rb(c, carry):
            pltpu.sync_copy(acc.at[idr.at[c]], stg)
            pltpu.sync_copy(stg, hp_ref.at[w * PRC + c])
            return carry

        lax.fori_loop(0, PRC, rb, 0)

    return hist


_hist0 = _make_hist(0)
_hist1 = _make_hist(1)


def _make_pool(half):
    lo = half * NHALF

    @functools.partial(
        pl.kernel,
        out_type=jax.ShapeDtypeStruct((NW * PRC, PRL, D), jnp.float32),
        scratch_types=[
            pltpu.VMEM((NPB, PB), jnp.int32),
            pltpu.VMEM((NPB, PB), jnp.int32),
            pltpu.VMEM((PB, D), jnp.float32),
            pltpu.VMEM((PB, D), jnp.float32),
            pltpu.VMEM((PRL, D), jnp.float32),
            pltpu.VMEM((PRC, PRL), jnp.int32),
            pltpu.VMEM_SHARED((NPH, D), jnp.float32),
            pltpu.SemaphoreType.DMA,
            pltpu.SemaphoreType.DMA,
            pltpu.SemaphoreType.DMA,
        ],
        mesh=_mesh(),
    )
    def pool(xn_ref, src_ref, dst_ref, p_ref, srcv, dstv, rbuf0, rbuf1, stg,
             idr, acc, gsem, ssem0, ssem1):
        cid = lax.axis_index("c")
        sid = lax.axis_index("s")
        w = cid * NS + sid
        pltpu.sync_copy(src_ref.at[w], srcv)
        pltpu.sync_copy(dst_ref.at[w], dstv)
        z16 = jnp.zeros((16,), jnp.float32)

        def tb(t, carry):
            j = t // (PB // 16)
            k = t % (PB // 16)
            sl = pl.ds(k * 16, 16)
            d = dstv[j, sl]
            dl = d - lo
            inh = (dl >= 0) & (dl < NHALF)
            dstv[j, sl] = jnp.where(inh, dl, NHALF + (d & 63))
            return carry

        lax.fori_loop(0, NPB * (PB // 16), tb, 0)

        def zb(i, carry):
            def zc(k, carry2):
                stg[i, pl.ds(k * 16, 16)] = z16
                return carry2

            lax.fori_loop(0, D // 16, zc, 0)
            return carry

        lax.fori_loop(0, PRL, zb, 0)
        _identity_rows(idr, sid * (PRC * PRL), PRC, PRL)

        def zs(c, carry):
            pltpu.sync_copy(stg, acc.at[idr.at[c]])
            return carry

        lax.fori_loop(0, PRC, zs, 0)
        plsc.subcore_barrier()

        def step(j, rbuf, ssem):
            @pl.when(j >= 2)
            def _():
                pltpu.make_async_copy(rbuf, acc.at[idr.at[0]], ssem).wait()

            pltpu.async_copy(xn_ref.at[srcv.at[j]], rbuf, gsem).wait()
            pltpu.async_copy(rbuf, acc.at[dstv.at[j]], ssem, add=True)

        def body(j, carry):
            @pl.when((j & 1) == 0)
            def _():
                step(j, rbuf0, ssem0)

            @pl.when((j & 1) == 1)
            def _():
                step(j, rbuf1, ssem1)

            return carry

        lax.fori_loop(0, NPB, body, 0)
        pltpu.make_async_copy(rbuf0, acc.at[idr.at[0]], ssem0).wait()
        pltpu.make_async_copy(rbuf1, acc.at[idr.at[0]], ssem1).wait()
        plsc.subcore_barrier()

        def rb(c, carry):
            pltpu.sync_copy(acc.at[idr.at[c]], stg)
            pltpu.sync_copy(stg, p_ref.at[w * PRC + c])
            return carry

        lax.fori_loop(0, PRC, rb, 0)

    return pool


_pool0 = _make_pool(0)
_pool1 = _make_pool(1)


def _invd_body(h00_ref, h01_ref, h10_ref, h11_ref, o_ref):
    d0 = h00_ref[...][:NHALF, :1] + h01_ref[...][:NHALF, :1]
    d1 = h10_ref[...][:NHALF, :1] + h11_ref[...][:NHALF, :1]
    o_ref[...] = lax.rsqrt(jnp.concatenate([d0, d1], axis=0))


def _invd_call(h00, h01, h10, h11):
    return pl.pallas_call(
        _invd_body,
        out_shape=jax.ShapeDtypeStruct((N, 1), jnp.float32),
    )(h00, h01, h10, h11)


def _scale_body(x_ref, iv_ref, o_ref):
    o_ref[...] = x_ref[...] * iv_ref[...]


def _out_body(p0_ref, p1_ref, iv_ref, w_ref, b_ref, o_ref):
    pooled = (p0_ref[...] + p1_ref[...]) * iv_ref[...]
    acc = jnp.dot(pooled, w_ref[...], preferred_element_type=jnp.float32)
    o_ref[...] = jnp.maximum(acc + b_ref[...], 0.0)


_RB = 2000


def _scale_call(x, iv):
    return pl.pallas_call(
        _scale_body,
        grid=(N // _RB,),
        in_specs=[
            pl.BlockSpec((_RB, D), lambda i: (i, 0)),
            pl.BlockSpec((_RB, 1), lambda i: (i, 0)),
        ],
        out_specs=pl.BlockSpec((_RB, D), lambda i: (i, 0)),
        out_shape=jax.ShapeDtypeStruct((N, D), jnp.float32),
    )(x, iv)


def _out_call(p0, p1, iv, W, b2):
    return pl.pallas_call(
        _out_body,
        grid=(N // _RB,),
        in_specs=[
            pl.BlockSpec((_RB, D), lambda i: (i, 0)),
            pl.BlockSpec((_RB, D), lambda i: (i, 0)),
            pl.BlockSpec((_RB, 1), lambda i: (i, 0)),
            pl.BlockSpec((D, D), lambda i: (0, 0)),
            pl.BlockSpec((1, D), lambda i: (0, 0)),
        ],
        out_specs=pl.BlockSpec((_RB, D), lambda i: (i, 0)),
        out_shape=jax.ShapeDtypeStruct((N, D), jnp.float32),
    )(p0, p1, iv, W, b2)


def kernel(x, edge_index, W, b):
    src = edge_index[0].astype(jnp.int32)
    dst = edge_index[1].astype(jnp.int32)
    src_p = src.reshape(NW, NPB, PB)
    dst_p = dst.reshape(NW, NPB, PB)

    hh0 = _hist0(dst_p).reshape(NC, NPH, D)
    hh1 = _hist1(dst_p).reshape(NC, NPH, D)
    iv = _invd_call(hh0[0], hh0[1], hh1[0], hh1[1])

    xn = _scale_call(x, iv)

    ph0 = _pool0(xn, src_p, dst_p).reshape(NC, NPH, D)
    ph1 = _pool1(xn, src_p, dst_p).reshape(NC, NPH, D)
    p0 = jnp.concatenate([ph0[0, :NHALF], ph1[0, :NHALF]], axis=0)
    p1 = jnp.concatenate([ph0[1, :NHALF], ph1[1, :NHALF]], axis=0)
    out = _out_call(p0, p1, iv, W, b.reshape(1, D))
    return out
